# baseline (speedup 1.0000x reference)
import functools

import jax
import jax.numpy as jnp
from jax import lax
from jax.experimental import pallas as pl
from jax.experimental.pallas import tpu as pltpu
from jax.experimental.pallas import tpu_sc as plsc

N = 10000
NP = 10240
E = 320000
EP = 327680
D = 128
NSUB = 16
NCORE = 2
EPW = EP // (NSUB * NCORE)
CH = 128
NCH = EPW // CH
ROWS_PT = NP // NSUB
RBLK = 512
GRID = NP // RBLK


def _sc_mesh():
    return plsc.VectorSubcoreMesh(core_axis_name="c", subcore_axis_name="s")


def _sc_degree(dst_p, zvec):
    @functools.partial(
        pl.kernel,
        out_type=jax.ShapeDtypeStruct((NCORE, NP), jnp.float32),
        mesh=_sc_mesh(),
        scratch_types=[
            pltpu.VMEM_SHARED((NP,), jnp.float32),
            pltpu.VMEM((NCH, CH), jnp.int32),
            pltpu.VMEM((CH,), jnp.float32),
        ],
    )
    def k(dst_hbm, z_hbm, out_hbm, dacc, dstv, ones_v):
        cid = lax.axis_index("c")
        sid = lax.axis_index("s")
        gw = cid * NSUB + sid
        for j in range(CH // 16):
            ones_v[pl.ds(j * 16, 16)] = jnp.full((16,), 1.0, jnp.float32)
        pltpu.sync_copy(dst_hbm.at[pl.ds(gw * NCH, NCH)], dstv)
        pltpu.sync_copy(z_hbm, dacc.at[pl.ds(sid * ROWS_PT, ROWS_PT)])
        plsc.subcore_barrier()

        def body(c, carry):
            pltpu.sync_copy(ones_v, dacc.at[dstv.at[c]], add=True)
            return carry

        lax.fori_loop(0, NCH, body, 0)
        plsc.subcore_barrier()
        pltpu.sync_copy(dacc.at[pl.ds(sid * ROWS_PT, ROWS_PT)],
                        out_hbm.at[cid, pl.ds(sid * ROWS_PT, ROWS_PT)])

    return k(dst_p, zvec)


def _sc_aggregate(y, src_p, dst_p, zrows):
    H = NCH // 2

    @functools.partial(
        pl.kernel,
        out_type=jax.ShapeDtypeStruct((NCORE, NP, D), jnp.float32),
        mesh=_sc_mesh(),
        scratch_types=[
            pltpu.VMEM_SHARED((NP, D), jnp.float32),
            pltpu.VMEM((H, CH), jnp.int32),
            pltpu.VMEM((H, CH), jnp.int32),
            pltpu.VMEM((2, CH, D), jnp.float32),
            pltpu.SemaphoreType.DMA((2,)),
            pltpu.SemaphoreType.DMA((2,)),
        ],
    )
    def k(y_hbm, src_hbm, dst_hbm, z_hbm, out_hbm, acc, srcv, dstv, rows,
          gs, ss):
        cid = lax.axis_index("c")
        sid = lax.axis_index("s")
        gw = cid * NSUB + sid
        pltpu.sync_copy(z_hbm, acc.at[pl.ds(sid * ROWS_PT, ROWS_PT)])
        plsc.subcore_barrier()

        def gstart(c, b):
            pltpu.async_copy(y_hbm.at[srcv.at[c]], rows.at[b], gs.at[b])

        def gwait(c, b):
            pltpu.make_async_copy(y_hbm.at[srcv.at[c]], rows.at[b],
                                  gs.at[b]).wait()

        def sstart(c, b):
            pltpu.async_copy(rows.at[b], acc.at[dstv.at[c]], ss.at[b],
                             add=True)

        def swait(c, b):
            pltpu.make_async_copy(rows.at[b], acc.at[dstv.at[c]],
                                  ss.at[b]).wait()

        def step(c, b):
            gwait(c, b)
            sstart(c, b)
            swait(c - 1, b ^ 1)
            gstart(c + 1, b ^ 1)

        for h in range(2):
            pltpu.sync_copy(src_hbm.at[pl.ds(gw * NCH + h * H, H)], srcv)
            pltpu.sync_copy(dst_hbm.at[pl.ds(gw * NCH + h * H, H)], dstv)
            gstart(0, 0)
            gstart(1, 1)
            gwait(0, 0)
            sstart(0, 0)

            def body(i, carry):
                c0 = 2 * i + 1
                step(c0, 1)
                step(c0 + 1, 0)
                return carry

            lax.fori_loop(0, (H - 2) // 2, body, 0)
            gwait(H - 1, 1)
            sstart(H - 1, 1)
            swait(H - 2, 0)
            swait(H - 1, 1)

        plsc.subcore_barrier()
        pltpu.sync_copy(acc.at[pl.ds(sid * ROWS_PT, ROWS_PT)],
                        out_hbm.at[cid, pl.ds(sid * ROWS_PT, ROWS_PT)])

    return k(y, src_p, dst_p, zrows)


def _dinv_from(dp_ref):
    deg = dp_ref[0, :] + dp_ref[1, :] + 1.0
    return lax.rsqrt(deg)[:, None]


def _tc_stage_a(x_p, W1, degp):
    def body(x_ref, w_ref, dp_ref, y_ref):
        xw = jnp.dot(x_ref[...], w_ref[...],
                     preferred_element_type=jnp.float32)
        y_ref[...] = xw * _dinv_from(dp_ref)

    return pl.pallas_call(
        body,
        grid=(GRID,),
        in_specs=[
            pl.BlockSpec((RBLK, D), lambda i: (i, 0)),
            pl.BlockSpec((D, D), lambda i: (0, 0)),
            pl.BlockSpec((NCORE, RBLK), lambda i: (0, i)),
        ],
        out_specs=pl.BlockSpec((RBLK, D), lambda i: (i, 0)),
        out_shape=jax.ShapeDtypeStruct((NP, D), jnp.float32),
    )(x_p, W1, degp)


def _tc_stage_c(p, y1, degp, W2, b1):
    def body(p_ref, y_ref, dp_ref, w_ref, b_ref, o_ref):
        dinv = _dinv_from(dp_ref)
        acc = p_ref[0] + p_ref[1] + y_ref[...]
        h = jnp.maximum(acc * dinv + b_ref[...], 0.0)
        o_ref[...] = jnp.dot(h, w_ref[...],
                             preferred_element_type=jnp.float32) * dinv

    return pl.pallas_call(
        body,
        grid=(GRID,),
        in_specs=[
            pl.BlockSpec((NCORE, RBLK, D), lambda i: (0, i, 0)),
            pl.BlockSpec((RBLK, D), lambda i: (i, 0)),
            pl.BlockSpec((NCORE, RBLK), lambda i: (0, i)),
            pl.BlockSpec((D, D), lambda i: (0, 0)),
            pl.BlockSpec((1, D), lambda i: (0, 0)),
        ],
        out_specs=pl.BlockSpec((RBLK, D), lambda i: (i, 0)),
        out_shape=jax.ShapeDtypeStruct((NP, D), jnp.float32),
    )(p, y1, degp, W2, b1)


def _tc_stage_e(q, y2, degp, b2):
    def body(q_ref, y_ref, dp_ref, b_ref, o_ref, l_ref):
        dinv = _dinv_from(dp_ref)
        out = (q_ref[0] + q_ref[1] + y_ref[...]) * dinv + b_ref[...]
        m = jnp.max(out, axis=1, keepdims=True)
        ex = jnp.exp(out - m)
        s = jnp.sum(ex, axis=1, keepdims=True)
        o_ref[...] = out
        l_ref[...] = out - m - jnp.log(s)

    return pl.pallas_call(
        body,
        grid=(GRID,),
        in_specs=[
            pl.BlockSpec((NCORE, RBLK, D), lambda i: (0, i, 0)),
            pl.BlockSpec((RBLK, D), lambda i: (i, 0)),
            pl.BlockSpec((NCORE, RBLK), lambda i: (0, i)),
            pl.BlockSpec((1, D), lambda i: (0, 0)),
        ],
        out_specs=[
            pl.BlockSpec((RBLK, D), lambda i: (i, 0)),
            pl.BlockSpec((RBLK, D), lambda i: (i, 0)),
        ],
        out_shape=[
            jax.ShapeDtypeStruct((NP, D), jnp.float32),
            jax.ShapeDtypeStruct((NP, D), jnp.float32),
        ],
    )(q, y2, degp, b2)


def kernel(x, edge_index, W1, b1, W2, b2):
    src = edge_index[0].astype(jnp.int32)
    dst = edge_index[1].astype(jnp.int32)
    pad_e = EP - E
    src_p = jnp.concatenate([src, jnp.zeros((pad_e,), jnp.int32)])
    pad_dst = N + (jnp.arange(pad_e, dtype=jnp.int32) % (NP - N))
    dst_p = jnp.concatenate([dst, pad_dst])
    src_p = src_p.reshape(EP // CH, CH)
    dst_p = dst_p.reshape(EP // CH, CH)
    x_p = jnp.zeros((NP, D), jnp.float32).at[:N].set(x)
    zvec = jnp.zeros((ROWS_PT,), jnp.float32)
    zrows = jnp.zeros((ROWS_PT, D), jnp.float32)

    degp = _sc_degree(dst_p, zvec)
    y1 = _tc_stage_a(x_p, W1, degp)
    p = _sc_aggregate(y1, src_p, dst_p, zrows)
    y2 = _tc_stage_c(p, y1, degp, W2, b1.reshape(1, D))
    q = _sc_aggregate(y2, src_p, dst_p, zrows)
    out, logp = _tc_stage_e(q, y2, degp, b2.reshape(1, D))
    return (out[:N], logp[:N])


# speedup vs baseline: 2.8647x; 2.8647x over previous
import functools

import jax
import jax.numpy as jnp
from jax import lax
from jax.experimental import pallas as pl
from jax.experimental.pallas import tpu as pltpu
from jax.experimental.pallas import tpu_sc as plsc

N = 10000
NP = 10240
E = 320000
EP = 327680
D = 128
NSUB = 16
NCORE = 2
EPW = EP // (NSUB * NCORE)
CH = 128
NCH = EPW // CH
ROWS_PT = NP // NSUB
RBLK = 512
GRID = NP // RBLK


def _sc_mesh():
    return plsc.VectorSubcoreMesh(core_axis_name="c", subcore_axis_name="s")


def _sc_degree(dst_p, zvec):
    @functools.partial(
        pl.kernel,
        out_type=jax.ShapeDtypeStruct((NCORE, NP), jnp.float32),
        mesh=_sc_mesh(),
        scratch_types=[
            pltpu.VMEM_SHARED((NP,), jnp.float32),
            pltpu.VMEM((NCH, CH), jnp.int32),
            pltpu.VMEM((CH,), jnp.float32),
        ],
    )
    def k(dst_hbm, z_hbm, out_hbm, dacc, dstv, ones_v):
        cid = lax.axis_index("c")
        sid = lax.axis_index("s")
        gw = cid * NSUB + sid
        for j in range(CH // 16):
            ones_v[pl.ds(j * 16, 16)] = jnp.full((16,), 1.0, jnp.float32)
        pltpu.sync_copy(dst_hbm.at[pl.ds(gw * NCH, NCH)], dstv)
        pltpu.sync_copy(z_hbm, dacc.at[pl.ds(sid * ROWS_PT, ROWS_PT)])
        plsc.subcore_barrier()

        def body(c, carry):
            pltpu.sync_copy(ones_v, dacc.at[dstv.at[c]], add=True)
            return carry

        lax.fori_loop(0, NCH, body, 0)
        plsc.subcore_barrier()
        pltpu.sync_copy(dacc.at[pl.ds(sid * ROWS_PT, ROWS_PT)],
                        out_hbm.at[cid, pl.ds(sid * ROWS_PT, ROWS_PT)])

    return k(dst_p, zvec)


def _sc_aggregate(y, src_p, dst_p, zrows):
    H = NCH // 2

    @functools.partial(
        pl.kernel,
        out_type=jax.ShapeDtypeStruct((NCORE, NP, D), jnp.float32),
        mesh=_sc_mesh(),
        scratch_types=[
            pltpu.VMEM_SHARED((NP, D), jnp.float32),
            pltpu.VMEM((H, CH), jnp.int32),
            pltpu.VMEM((H, CH), jnp.int32),
            pltpu.VMEM((2, CH, D), jnp.float32),
            pltpu.SemaphoreType.DMA((2,)),
            pltpu.SemaphoreType.DMA((2,)),
        ],
    )
    def k(y_hbm, src_hbm, dst_hbm, z_hbm, out_hbm, acc, srcv, dstv, rows,
          gs, ss):
        cid = lax.axis_index("c")
        sid = lax.axis_index("s")
        gw = cid * NSUB + sid
        pltpu.sync_copy(z_hbm, acc.at[pl.ds(sid * ROWS_PT, ROWS_PT)])
        plsc.subcore_barrier()

        def gstart(c, b):
            pltpu.async_copy(y_hbm.at[srcv.at[c]], rows.at[b], gs.at[b])

        def gwait(c, b):
            pltpu.make_async_copy(y_hbm.at[srcv.at[c]], rows.at[b],
                                  gs.at[b]).wait()

        def sstart(c, b):
            pltpu.async_copy(rows.at[b], acc.at[dstv.at[c]], ss.at[b],
                             add=True)

        def swait(c, b):
            pltpu.make_async_copy(rows.at[b], acc.at[dstv.at[c]],
                                  ss.at[b]).wait()

        def step(c, b):
            gwait(c, b)
            sstart(c, b)
            swait(c - 1, b ^ 1)
            gstart(c + 1, b ^ 1)

        for h in range(2):
            pltpu.sync_copy(src_hbm.at[pl.ds(gw * NCH + h * H, H)], srcv)
            pltpu.sync_copy(dst_hbm.at[pl.ds(gw * NCH + h * H, H)], dstv)
            gstart(0, 0)
            gstart(1, 1)
            gwait(0, 0)
            sstart(0, 0)

            def body(i, carry):
                c0 = 2 * i + 1
                step(c0, 1)
                step(c0 + 1, 0)
                return carry

            lax.fori_loop(0, (H - 2) // 2, body, 0)
            gwait(H - 1, 1)
            sstart(H - 1, 1)
            swait(H - 2, 0)
            swait(H - 1, 1)

        plsc.subcore_barrier()
        pltpu.sync_copy(acc.at[pl.ds(sid * ROWS_PT, ROWS_PT)],
                        out_hbm.at[cid, pl.ds(sid * ROWS_PT, ROWS_PT)])

    return k(y, src_p, dst_p, zrows)


def _dinv_from(dp_ref):
    deg = dp_ref[0, :] + dp_ref[1, :] + 1.0
    return lax.rsqrt(deg)[:, None]


def _tc_stage_a(x_p, W1, degp):
    def body(x_ref, w_ref, dp_ref, y_ref):
        xw = jnp.dot(x_ref[...], w_ref[...],
                     preferred_element_type=jnp.float32)
        y_ref[...] = xw * _dinv_from(dp_ref)

    return pl.pallas_call(
        body,
        grid=(GRID,),
        in_specs=[
            pl.BlockSpec((RBLK, D), lambda i: (i, 0)),
            pl.BlockSpec((D, D), lambda i: (0, 0)),
            pl.BlockSpec((NCORE, RBLK), lambda i: (0, i)),
        ],
        out_specs=pl.BlockSpec((RBLK, D), lambda i: (i, 0)),
        out_shape=jax.ShapeDtypeStruct((NP, D), jnp.float32),
    )(x_p, W1, degp)


def _tc_stage_c(p, y1, degp, W2, b1):
    def body(p_ref, y_ref, dp_ref, w_ref, b_ref, o_ref):
        dinv = _dinv_from(dp_ref)
        acc = p_ref[0] + p_ref[1] + y_ref[...]
        h = jnp.maximum(acc * dinv + b_ref[...], 0.0)
        o_ref[...] = jnp.dot(h, w_ref[...],
                             preferred_element_type=jnp.float32) * dinv

    return pl.pallas_call(
        body,
        grid=(GRID,),
        in_specs=[
            pl.BlockSpec((NCORE, RBLK, D), lambda i: (0, i, 0)),
            pl.BlockSpec((RBLK, D), lambda i: (i, 0)),
            pl.BlockSpec((NCORE, RBLK), lambda i: (0, i)),
            pl.BlockSpec((D, D), lambda i: (0, 0)),
            pl.BlockSpec((1, D), lambda i: (0, 0)),
        ],
        out_specs=pl.BlockSpec((RBLK, D), lambda i: (i, 0)),
        out_shape=jax.ShapeDtypeStruct((NP, D), jnp.float32),
    )(p, y1, degp, W2, b1)


def _tc_stage_e(q, y2, degp, b2):
    def body(q_ref, y_ref, dp_ref, b_ref, o_ref, l_ref):
        dinv = _dinv_from(dp_ref)
        out = (q_ref[0] + q_ref[1] + y_ref[...]) * dinv + b_ref[...]
        m = jnp.max(out, axis=1, keepdims=True)
        ex = jnp.exp(out - m)
        s = jnp.sum(ex, axis=1, keepdims=True)
        o_ref[...] = out
        l_ref[...] = out - m - jnp.log(s)

    return pl.pallas_call(
        body,
        grid=(GRID,),
        in_specs=[
            pl.BlockSpec((NCORE, RBLK, D), lambda i: (0, i, 0)),
            pl.BlockSpec((RBLK, D), lambda i: (i, 0)),
            pl.BlockSpec((NCORE, RBLK), lambda i: (0, i)),
            pl.BlockSpec((1, D), lambda i: (0, 0)),
        ],
        out_specs=[
            pl.BlockSpec((RBLK, D), lambda i: (i, 0)),
            pl.BlockSpec((RBLK, D), lambda i: (i, 0)),
        ],
        out_shape=[
            jax.ShapeDtypeStruct((NP, D), jnp.float32),
            jax.ShapeDtypeStruct((NP, D), jnp.float32),
        ],
    )(q, y2, degp, b2)


def kernel(x, edge_index, W1, b1, W2, b2):
    src = edge_index[0].astype(jnp.int32)
    dst = edge_index[1].astype(jnp.int32)
    pad_e = EP - E
    pad_src = jnp.arange(pad_e, dtype=jnp.int32) % N
    src_p = jnp.concatenate([src, pad_src])
    pad_dst = N + (jnp.arange(pad_e, dtype=jnp.int32) % (NP - N))
    dst_p = jnp.concatenate([dst, pad_dst])
    src_p = src_p.reshape(EP // CH, CH)
    dst_p = dst_p.reshape(EP // CH, CH)
    x_p = jnp.zeros((NP, D), jnp.float32).at[:N].set(x)
    zvec = jnp.zeros((ROWS_PT,), jnp.float32)
    zrows = jnp.zeros((ROWS_PT, D), jnp.float32)

    degp = _sc_degree(dst_p, zvec)
    y1 = _tc_stage_a(x_p, W1, degp)
    p = _sc_aggregate(y1, src_p, dst_p, zrows)
    y2 = _tc_stage_c(p, y1, degp, W2, b1.reshape(1, D))
    q = _sc_aggregate(y2, src_p, dst_p, zrows)
    out, logp = _tc_stage_e(q, y2, degp, b2.reshape(1, D))
    return (out[:N], logp[:N])


# speedup vs baseline: 2.9317x; 1.0234x over previous
import functools

import jax
import jax.numpy as jnp
from jax import lax
from jax.experimental import pallas as pl
from jax.experimental.pallas import tpu as pltpu
from jax.experimental.pallas import tpu_sc as plsc

N = 10000
NP = 10240
E = 320000
EP = 327680
D = 128
NSUB = 16
NCORE = 2
EPW = EP // (NSUB * NCORE)
CH = 128
NCH = EPW // CH
ROWS_PT = NP // NSUB
RBLK = 512
GRID = (N + RBLK - 1) // RBLK


def _sc_mesh():
    return plsc.VectorSubcoreMesh(core_axis_name="c", subcore_axis_name="s")


def _sc_degree(dst_p, zvec):
    @functools.partial(
        pl.kernel,
        out_type=jax.ShapeDtypeStruct((NCORE, NP), jnp.float32),
        mesh=_sc_mesh(),
        scratch_types=[
            pltpu.VMEM_SHARED((NP,), jnp.float32),
            pltpu.VMEM((NCH, CH), jnp.int32),
            pltpu.VMEM((CH,), jnp.float32),
        ],
    )
    def k(dst_hbm, z_hbm, out_hbm, dacc, dstv, ones_v):
        cid = lax.axis_index("c")
        sid = lax.axis_index("s")
        gw = cid * NSUB + sid
        for j in range(CH // 16):
            ones_v[pl.ds(j * 16, 16)] = jnp.full((16,), 1.0, jnp.float32)
        pltpu.sync_copy(dst_hbm.at[pl.ds(gw * NCH, NCH)], dstv)
        pltpu.sync_copy(z_hbm, dacc.at[pl.ds(sid * ROWS_PT, ROWS_PT)])
        plsc.subcore_barrier()

        def body(c, carry):
            pltpu.sync_copy(ones_v, dacc.at[dstv.at[c]], add=True)
            return carry

        lax.fori_loop(0, NCH, body, 0)
        plsc.subcore_barrier()
        pltpu.sync_copy(dacc.at[pl.ds(sid * ROWS_PT, ROWS_PT)],
                        out_hbm.at[cid, pl.ds(sid * ROWS_PT, ROWS_PT)])

    return k(dst_p, zvec)


def _sc_aggregate(y, src_p, dst_p, zrows):
    H = NCH // 2

    @functools.partial(
        pl.kernel,
        out_type=jax.ShapeDtypeStruct((NCORE, NP, D), jnp.float32),
        mesh=_sc_mesh(),
        scratch_types=[
            pltpu.VMEM_SHARED((NP, D), jnp.float32),
            pltpu.VMEM((H, CH), jnp.int32),
            pltpu.VMEM((H, CH), jnp.int32),
            pltpu.VMEM((2, CH, D), jnp.float32),
            pltpu.SemaphoreType.DMA((2,)),
            pltpu.SemaphoreType.DMA((2,)),
        ],
    )
    def k(y_hbm, src_hbm, dst_hbm, z_hbm, out_hbm, acc, srcv, dstv, rows,
          gs, ss):
        cid = lax.axis_index("c")
        sid = lax.axis_index("s")
        gw = cid * NSUB + sid
        pltpu.sync_copy(z_hbm, acc.at[pl.ds(sid * ROWS_PT, ROWS_PT)])
        plsc.subcore_barrier()

        def gstart(c, b):
            pltpu.async_copy(y_hbm.at[srcv.at[c]], rows.at[b], gs.at[b])

        def gwait(c, b):
            pltpu.make_async_copy(y_hbm.at[srcv.at[c]], rows.at[b],
                                  gs.at[b]).wait()

        def sstart(c, b):
            pltpu.async_copy(rows.at[b], acc.at[dstv.at[c]], ss.at[b],
                             add=True)

        def swait(c, b):
            pltpu.make_async_copy(rows.at[b], acc.at[dstv.at[c]],
                                  ss.at[b]).wait()

        def step(c, b):
            gwait(c, b)
            sstart(c, b)
            swait(c - 1, b ^ 1)
            gstart(c + 1, b ^ 1)

        for h in range(2):
            pltpu.sync_copy(src_hbm.at[pl.ds(gw * NCH + h * H, H)], srcv)
            pltpu.sync_copy(dst_hbm.at[pl.ds(gw * NCH + h * H, H)], dstv)
            gstart(0, 0)
            gstart(1, 1)
            gwait(0, 0)
            sstart(0, 0)

            def body(i, carry):
                c0 = 2 * i + 1
                step(c0, 1)
                step(c0 + 1, 0)
                return carry

            lax.fori_loop(0, (H - 2) // 2, body, 0)
            gwait(H - 1, 1)
            sstart(H - 1, 1)
            swait(H - 2, 0)
            swait(H - 1, 1)

        plsc.subcore_barrier()
        pltpu.sync_copy(acc.at[pl.ds(sid * ROWS_PT, ROWS_PT)],
                        out_hbm.at[cid, pl.ds(sid * ROWS_PT, ROWS_PT)])

    return k(y, src_p, dst_p, zrows)


def _dinv_from(dp_ref):
    deg = dp_ref[0, :] + dp_ref[1, :] + 1.0
    return lax.rsqrt(deg)[:, None]


def _tc_stage_a(x_p, W1, degp):
    def body(x_ref, w_ref, dp_ref, y_ref):
        xw = jnp.dot(x_ref[...], w_ref[...],
                     preferred_element_type=jnp.float32)
        y_ref[...] = xw * _dinv_from(dp_ref)

    return pl.pallas_call(
        body,
        grid=(GRID,),
        in_specs=[
            pl.BlockSpec((RBLK, D), lambda i: (i, 0)),
            pl.BlockSpec((D, D), lambda i: (0, 0)),
            pl.BlockSpec((NCORE, RBLK), lambda i: (0, i)),
        ],
        out_specs=pl.BlockSpec((RBLK, D), lambda i: (i, 0)),
        out_shape=jax.ShapeDtypeStruct((N, D), jnp.float32),
    )(x_p, W1, degp)


def _tc_stage_c(p, y1, degp, W2, b1):
    def body(p_ref, y_ref, dp_ref, w_ref, b_ref, o_ref):
        dinv = _dinv_from(dp_ref)
        acc = p_ref[0] + p_ref[1] + y_ref[...]
        h = jnp.maximum(acc * dinv + b_ref[...], 0.0)
        o_ref[...] = jnp.dot(h, w_ref[...],
                             preferred_element_type=jnp.float32) * dinv

    return pl.pallas_call(
        body,
        grid=(GRID,),
        in_specs=[
            pl.BlockSpec((NCORE, RBLK, D), lambda i: (0, i, 0)),
            pl.BlockSpec((RBLK, D), lambda i: (i, 0)),
            pl.BlockSpec((NCORE, RBLK), lambda i: (0, i)),
            pl.BlockSpec((D, D), lambda i: (0, 0)),
            pl.BlockSpec((1, D), lambda i: (0, 0)),
        ],
        out_specs=pl.BlockSpec((RBLK, D), lambda i: (i, 0)),
        out_shape=jax.ShapeDtypeStruct((N, D), jnp.float32),
    )(p, y1, degp, W2, b1)


def _tc_stage_e(q, y2, degp, b2):
    def body(q_ref, y_ref, dp_ref, b_ref, o_ref, l_ref):
        dinv = _dinv_from(dp_ref)
        out = (q_ref[0] + q_ref[1] + y_ref[...]) * dinv + b_ref[...]
        m = jnp.max(out, axis=1, keepdims=True)
        ex = jnp.exp(out - m)
        s = jnp.sum(ex, axis=1, keepdims=True)
        o_ref[...] = out
        l_ref[...] = out - m - jnp.log(s)

    return pl.pallas_call(
        body,
        grid=(GRID,),
        in_specs=[
            pl.BlockSpec((NCORE, RBLK, D), lambda i: (0, i, 0)),
            pl.BlockSpec((RBLK, D), lambda i: (i, 0)),
            pl.BlockSpec((NCORE, RBLK), lambda i: (0, i)),
            pl.BlockSpec((1, D), lambda i: (0, 0)),
        ],
        out_specs=[
            pl.BlockSpec((RBLK, D), lambda i: (i, 0)),
            pl.BlockSpec((RBLK, D), lambda i: (i, 0)),
        ],
        out_shape=[
            jax.ShapeDtypeStruct((N, D), jnp.float32),
            jax.ShapeDtypeStruct((N, D), jnp.float32),
        ],
    )(q, y2, degp, b2)


def kernel(x, edge_index, W1, b1, W2, b2):
    src = edge_index[0].astype(jnp.int32)
    dst = edge_index[1].astype(jnp.int32)
    pad_e = EP - E
    pad_src = jnp.arange(pad_e, dtype=jnp.int32) % N
    src_p = jnp.concatenate([src, pad_src])
    pad_dst = N + (jnp.arange(pad_e, dtype=jnp.int32) % (NP - N))
    dst_p = jnp.concatenate([dst, pad_dst])
    src_p = src_p.reshape(EP // CH, CH)
    dst_p = dst_p.reshape(EP // CH, CH)
    zvec = jnp.zeros((ROWS_PT,), jnp.float32)
    zrows = jnp.zeros((ROWS_PT, D), jnp.float32)

    degp = _sc_degree(dst_p, zvec)
    y1 = _tc_stage_a(x, W1, degp)
    p = _sc_aggregate(y1, src_p, dst_p, zrows)
    y2 = _tc_stage_c(p, y1, degp, W2, b1.reshape(1, D))
    q = _sc_aggregate(y2, src_p, dst_p, zrows)
    out, logp = _tc_stage_e(q, y2, degp, b2.reshape(1, D))
    return (out, logp)


# speedup vs baseline: 2.9935x; 1.0211x over previous
import functools

import jax
import jax.numpy as jnp
from jax import lax
from jax.experimental import pallas as pl
from jax.experimental.pallas import tpu as pltpu
from jax.experimental.pallas import tpu_sc as plsc

N = 10000
NP = 10240
E = 320000
EP = 327680
D = 128
NSUB = 16
NCORE = 2
EPW = EP // (NSUB * NCORE)
CH = 128
NCH = EPW // CH
ROWS_PT = NP // NSUB
RBLK = 512
GRID = (N + RBLK - 1) // RBLK


def _sc_mesh():
    return plsc.VectorSubcoreMesh(core_axis_name="c", subcore_axis_name="s")


def _sc_degree(dst_p, zvec):
    @functools.partial(
        pl.kernel,
        out_type=jax.ShapeDtypeStruct((NCORE, NP), jnp.float32),
        mesh=_sc_mesh(),
        scratch_types=[
            pltpu.VMEM_SHARED((NP,), jnp.float32),
            pltpu.VMEM((NCH, CH), jnp.int32),
            pltpu.VMEM((CH,), jnp.float32),
        ],
    )
    def k(dst_hbm, z_hbm, out_hbm, dacc, dstv, ones_v):
        cid = lax.axis_index("c")
        sid = lax.axis_index("s")
        gw = cid * NSUB + sid
        for j in range(CH // 16):
            ones_v[pl.ds(j * 16, 16)] = jnp.full((16,), 1.0, jnp.float32)
        pltpu.sync_copy(dst_hbm.at[pl.ds(gw * NCH, NCH)], dstv)
        pltpu.sync_copy(z_hbm, dacc.at[pl.ds(sid * ROWS_PT, ROWS_PT)])
        plsc.subcore_barrier()

        def body(c, carry):
            pltpu.sync_copy(ones_v, dacc.at[dstv.at[c]], add=True)
            return carry

        lax.fori_loop(0, NCH, body, 0)
        plsc.subcore_barrier()
        pltpu.sync_copy(dacc.at[pl.ds(sid * ROWS_PT, ROWS_PT)],
                        out_hbm.at[cid, pl.ds(sid * ROWS_PT, ROWS_PT)])

    return k(dst_p, zvec)


ACH = 64
ANCH = EPW // ACH
AH = ANCH // 4
NPRE = 4


def _sc_aggregate(y, src_p, dst_p, zrows):
    @functools.partial(
        pl.kernel,
        out_type=jax.ShapeDtypeStruct((NCORE, NP, D), jnp.float32),
        mesh=_sc_mesh(),
        scratch_types=[
            pltpu.VMEM_SHARED((NP, D), jnp.float32),
            pltpu.VMEM((AH, ACH), jnp.int32),
            pltpu.VMEM((AH, ACH), jnp.int32),
            pltpu.VMEM((4, ACH, D), jnp.float32),
            pltpu.SemaphoreType.DMA((4,)),
            pltpu.SemaphoreType.DMA((4,)),
        ],
    )
    def k(y_hbm, src_hbm, dst_hbm, z_hbm, out_hbm, acc, srcv, dstv, rows,
          gs, ss):
        cid = lax.axis_index("c")
        sid = lax.axis_index("s")
        gw = cid * NSUB + sid
        pltpu.sync_copy(z_hbm, acc.at[pl.ds(sid * ROWS_PT, ROWS_PT)])
        plsc.subcore_barrier()

        def gstart(c, b):
            pltpu.async_copy(y_hbm.at[srcv.at[c]], rows.at[b], gs.at[b])

        def gwait(c, b):
            pltpu.make_async_copy(y_hbm.at[srcv.at[c]], rows.at[b],
                                  gs.at[b]).wait()

        def sstart(c, b):
            pltpu.async_copy(rows.at[b], acc.at[dstv.at[c]], ss.at[b],
                             add=True)

        def swait(c, b):
            pltpu.make_async_copy(rows.at[b], acc.at[dstv.at[c]],
                                  ss.at[b]).wait()

        def step(c, b):
            gwait(c, b)
            sstart(c, b)
            swait(c - 2, (b + 2) & 3)
            gstart(c + 2, (b + 2) & 3)

        for h in range(NPRE):
            pltpu.sync_copy(src_hbm.at[pl.ds(gw * ANCH + h * AH, AH)], srcv)
            pltpu.sync_copy(dst_hbm.at[pl.ds(gw * ANCH + h * AH, AH)], dstv)
            gstart(0, 0)
            gstart(1, 1)
            gwait(0, 0); sstart(0, 0); gstart(2, 2)
            gwait(1, 1); sstart(1, 1); gstart(3, 3)

            def body(i, carry):
                c0 = 4 * i + 2
                step(c0 + 0, 2)
                step(c0 + 1, 3)
                step(c0 + 2, 0)
                step(c0 + 3, 1)
                return carry

            lax.fori_loop(0, (AH - 4) // 4, body, 0)
            gwait(AH - 2, 2); sstart(AH - 2, 2)
            gwait(AH - 1, 3); sstart(AH - 1, 3)
            swait(AH - 4, 0)
            swait(AH - 3, 1)
            swait(AH - 2, 2)
            swait(AH - 1, 3)

        plsc.subcore_barrier()
        pltpu.sync_copy(acc.at[pl.ds(sid * ROWS_PT, ROWS_PT)],
                        out_hbm.at[cid, pl.ds(sid * ROWS_PT, ROWS_PT)])

    return k(y, src_p, dst_p, zrows)


def _dinv_from(dp_ref):
    deg = dp_ref[0, :] + dp_ref[1, :] + 1.0
    return lax.rsqrt(deg)[:, None]


def _tc_stage_a(x_p, W1, degp):
    def body(x_ref, w_ref, dp_ref, y_ref):
        xw = jnp.dot(x_ref[...], w_ref[...],
                     preferred_element_type=jnp.float32)
        y_ref[...] = xw * _dinv_from(dp_ref)

    return pl.pallas_call(
        body,
        grid=(GRID,),
        in_specs=[
            pl.BlockSpec((RBLK, D), lambda i: (i, 0)),
            pl.BlockSpec((D, D), lambda i: (0, 0)),
            pl.BlockSpec((NCORE, RBLK), lambda i: (0, i)),
        ],
        out_specs=pl.BlockSpec((RBLK, D), lambda i: (i, 0)),
        out_shape=jax.ShapeDtypeStruct((N, D), jnp.float32),
    )(x_p, W1, degp)


def _tc_stage_c(p, y1, degp, W2, b1):
    def body(p_ref, y_ref, dp_ref, w_ref, b_ref, o_ref):
        dinv = _dinv_from(dp_ref)
        acc = p_ref[0] + p_ref[1] + y_ref[...]
        h = jnp.maximum(acc * dinv + b_ref[...], 0.0)
        o_ref[...] = jnp.dot(h, w_ref[...],
                             preferred_element_type=jnp.float32) * dinv

    return pl.pallas_call(
        body,
        grid=(GRID,),
        in_specs=[
            pl.BlockSpec((NCORE, RBLK, D), lambda i: (0, i, 0)),
            pl.BlockSpec((RBLK, D), lambda i: (i, 0)),
            pl.BlockSpec((NCORE, RBLK), lambda i: (0, i)),
            pl.BlockSpec((D, D), lambda i: (0, 0)),
            pl.BlockSpec((1, D), lambda i: (0, 0)),
        ],
        out_specs=pl.BlockSpec((RBLK, D), lambda i: (i, 0)),
        out_shape=jax.ShapeDtypeStruct((N, D), jnp.float32),
    )(p, y1, degp, W2, b1)


def _tc_stage_e(q, y2, degp, b2):
    def body(q_ref, y_ref, dp_ref, b_ref, o_ref, l_ref):
        dinv = _dinv_from(dp_ref)
        out = (q_ref[0] + q_ref[1] + y_ref[...]) * dinv + b_ref[...]
        m = jnp.max(out, axis=1, keepdims=True)
        ex = jnp.exp(out - m)
        s = jnp.sum(ex, axis=1, keepdims=True)
        o_ref[...] = out
        l_ref[...] = out - m - jnp.log(s)

    return pl.pallas_call(
        body,
        grid=(GRID,),
        in_specs=[
            pl.BlockSpec((NCORE, RBLK, D), lambda i: (0, i, 0)),
            pl.BlockSpec((RBLK, D), lambda i: (i, 0)),
            pl.BlockSpec((NCORE, RBLK), lambda i: (0, i)),
            pl.BlockSpec((1, D), lambda i: (0, 0)),
        ],
        out_specs=[
            pl.BlockSpec((RBLK, D), lambda i: (i, 0)),
            pl.BlockSpec((RBLK, D), lambda i: (i, 0)),
        ],
        out_shape=[
            jax.ShapeDtypeStruct((N, D), jnp.float32),
            jax.ShapeDtypeStruct((N, D), jnp.float32),
        ],
    )(q, y2, degp, b2)


def kernel(x, edge_index, W1, b1, W2, b2):
    src = edge_index[0].astype(jnp.int32)
    dst = edge_index[1].astype(jnp.int32)
    pad_e = EP - E
    pad_src = jnp.arange(pad_e, dtype=jnp.int32) % N
    src_p = jnp.concatenate([src, pad_src])
    pad_dst = N + (jnp.arange(pad_e, dtype=jnp.int32) % (NP - N))
    dst_p = jnp.concatenate([dst, pad_dst])
    dst128 = dst_p.reshape(EP // CH, CH)
    src_p = src_p.reshape(EP // ACH, ACH)
    dst_p = dst_p.reshape(EP // ACH, ACH)
    zvec = jnp.zeros((ROWS_PT,), jnp.float32)
    zrows = jnp.zeros((ROWS_PT, D), jnp.float32)

    degp = _sc_degree(dst128, zvec)
    y1 = _tc_stage_a(x, W1, degp)
    p = _sc_aggregate(y1, src_p, dst_p, zrows)
    y2 = _tc_stage_c(p, y1, degp, W2, b1.reshape(1, D))
    q = _sc_aggregate(y2, src_p, dst_p, zrows)
    out, logp = _tc_stage_e(q, y2, degp, b2.reshape(1, D))
    return (out, logp)


# speedup vs baseline: 3.1814x; 1.0628x over previous
import functools

import jax
import jax.numpy as jnp
from jax import lax
from jax.experimental import pallas as pl
from jax.experimental.pallas import tpu as pltpu
from jax.experimental.pallas import tpu_sc as plsc

N = 10000
NP = 10240
E = 320000
EP = 327680
D = 128
NSUB = 16
NCORE = 2
EPW = EP // (NSUB * NCORE)
CH = 128
NCH = EPW // CH
ROWS_PT = NP // NSUB
RBLK = 2048
GRID = (N + RBLK - 1) // RBLK


def _sc_mesh():
    return plsc.VectorSubcoreMesh(core_axis_name="c", subcore_axis_name="s")


def _sc_degree(dst_p, zvec):
    @functools.partial(
        pl.kernel,
        out_type=jax.ShapeDtypeStruct((NCORE, NP), jnp.float32),
        mesh=_sc_mesh(),
        scratch_types=[
            pltpu.VMEM_SHARED((NP,), jnp.float32),
            pltpu.VMEM((NCH, CH), jnp.int32),
            pltpu.VMEM((CH,), jnp.float32),
        ],
    )
    def k(dst_hbm, z_hbm, out_hbm, dacc, dstv, ones_v):
        cid = lax.axis_index("c")
        sid = lax.axis_index("s")
        gw = cid * NSUB + sid
        for j in range(CH // 16):
            ones_v[pl.ds(j * 16, 16)] = jnp.full((16,), 1.0, jnp.float32)
        pltpu.sync_copy(dst_hbm.at[pl.ds(gw * NCH, NCH)], dstv)
        pltpu.sync_copy(z_hbm, dacc.at[pl.ds(sid * ROWS_PT, ROWS_PT)])
        plsc.subcore_barrier()

        def body(c, carry):
            pltpu.sync_copy(ones_v, dacc.at[dstv.at[c]], add=True)
            return carry

        lax.fori_loop(0, NCH, body, 0)
        plsc.subcore_barrier()
        pltpu.sync_copy(dacc.at[pl.ds(sid * ROWS_PT, ROWS_PT)],
                        out_hbm.at[cid, pl.ds(sid * ROWS_PT, ROWS_PT)])

    return k(dst_p, zvec)


ACH = 64
ANCH = EPW // ACH
AH = ANCH // 4
NPRE = 4


def _sc_aggregate(y, src_p, dst_p, zrows):
    @functools.partial(
        pl.kernel,
        out_type=jax.ShapeDtypeStruct((NCORE, NP, D), jnp.float32),
        mesh=_sc_mesh(),
        scratch_types=[
            pltpu.VMEM_SHARED((NP, D), jnp.float32),
            pltpu.VMEM((AH, ACH), jnp.int32),
            pltpu.VMEM((AH, ACH), jnp.int32),
            pltpu.VMEM((4, ACH, D), jnp.float32),
            pltpu.SemaphoreType.DMA((4,)),
            pltpu.SemaphoreType.DMA((4,)),
        ],
    )
    def k(y_hbm, src_hbm, dst_hbm, z_hbm, out_hbm, acc, srcv, dstv, rows,
          gs, ss):
        cid = lax.axis_index("c")
        sid = lax.axis_index("s")
        gw = cid * NSUB + sid
        pltpu.sync_copy(z_hbm, acc.at[pl.ds(sid * ROWS_PT, ROWS_PT)])
        plsc.subcore_barrier()

        def gstart(c, b):
            pltpu.async_copy(y_hbm.at[srcv.at[c]], rows.at[b], gs.at[b])

        def gwait(c, b):
            pltpu.make_async_copy(y_hbm.at[srcv.at[c]], rows.at[b],
                                  gs.at[b]).wait()

        def sstart(c, b):
            pltpu.async_copy(rows.at[b], acc.at[dstv.at[c]], ss.at[b],
                             add=True)

        def swait(c, b):
            pltpu.make_async_copy(rows.at[b], acc.at[dstv.at[c]],
                                  ss.at[b]).wait()

        def step(c, b):
            gwait(c, b)
            sstart(c, b)
            swait(c - 2, (b + 2) & 3)
            gstart(c + 2, (b + 2) & 3)

        for h in range(NPRE):
            pltpu.sync_copy(src_hbm.at[pl.ds(gw * ANCH + h * AH, AH)], srcv)
            pltpu.sync_copy(dst_hbm.at[pl.ds(gw * ANCH + h * AH, AH)], dstv)
            gstart(0, 0)
            gstart(1, 1)
            gwait(0, 0); sstart(0, 0); gstart(2, 2)
            gwait(1, 1); sstart(1, 1); gstart(3, 3)

            def body(i, carry):
                c0 = 4 * i + 2
                step(c0 + 0, 2)
                step(c0 + 1, 3)
                step(c0 + 2, 0)
                step(c0 + 3, 1)
                return carry

            lax.fori_loop(0, (AH - 4) // 4, body, 0)
            gwait(AH - 2, 2); sstart(AH - 2, 2)
            gwait(AH - 1, 3); sstart(AH - 1, 3)
            swait(AH - 4, 0)
            swait(AH - 3, 1)
            swait(AH - 2, 2)
            swait(AH - 1, 3)

        plsc.subcore_barrier()
        pltpu.sync_copy(acc.at[pl.ds(sid * ROWS_PT, ROWS_PT)],
                        out_hbm.at[cid, pl.ds(sid * ROWS_PT, ROWS_PT)])

    return k(y, src_p, dst_p, zrows)


def _dinv_from(dp_ref):
    deg = dp_ref[0, :] + dp_ref[1, :] + 1.0
    return lax.rsqrt(deg)[:, None]


def _tc_stage_a(x_p, W1, degp):
    def body(x_ref, w_ref, dp_ref, y_ref):
        xw = jnp.dot(x_ref[...], w_ref[...],
                     preferred_element_type=jnp.float32)
        y_ref[...] = xw * _dinv_from(dp_ref)

    return pl.pallas_call(
        body,
        grid=(GRID,),
        in_specs=[
            pl.BlockSpec((RBLK, D), lambda i: (i, 0)),
            pl.BlockSpec((D, D), lambda i: (0, 0)),
            pl.BlockSpec((NCORE, RBLK), lambda i: (0, i)),
        ],
        out_specs=pl.BlockSpec((RBLK, D), lambda i: (i, 0)),
        out_shape=jax.ShapeDtypeStruct((N, D), jnp.float32),
    )(x_p, W1, degp)


def _tc_stage_c(p, y1, degp, W2, b1):
    def body(p_ref, y_ref, dp_ref, w_ref, b_ref, o_ref):
        dinv = _dinv_from(dp_ref)
        acc = p_ref[0] + p_ref[1] + y_ref[...]
        h = jnp.maximum(acc * dinv + b_ref[...], 0.0)
        o_ref[...] = jnp.dot(h, w_ref[...],
                             preferred_element_type=jnp.float32) * dinv

    return pl.pallas_call(
        body,
        grid=(GRID,),
        in_specs=[
            pl.BlockSpec((NCORE, RBLK, D), lambda i: (0, i, 0)),
            pl.BlockSpec((RBLK, D), lambda i: (i, 0)),
            pl.BlockSpec((NCORE, RBLK), lambda i: (0, i)),
            pl.BlockSpec((D, D), lambda i: (0, 0)),
            pl.BlockSpec((1, D), lambda i: (0, 0)),
        ],
        out_specs=pl.BlockSpec((RBLK, D), lambda i: (i, 0)),
        out_shape=jax.ShapeDtypeStruct((N, D), jnp.float32),
    )(p, y1, degp, W2, b1)


def _tc_stage_e(q, y2, degp, b2):
    def body(q_ref, y_ref, dp_ref, b_ref, o_ref, l_ref):
        dinv = _dinv_from(dp_ref)
        out = (q_ref[0] + q_ref[1] + y_ref[...]) * dinv + b_ref[...]
        m = jnp.max(out, axis=1, keepdims=True)
        ex = jnp.exp(out - m)
        s = jnp.sum(ex, axis=1, keepdims=True)
        o_ref[...] = out
        l_ref[...] = out - m - jnp.log(s)

    return pl.pallas_call(
        body,
        grid=(GRID,),
        in_specs=[
            pl.BlockSpec((NCORE, RBLK, D), lambda i: (0, i, 0)),
            pl.BlockSpec((RBLK, D), lambda i: (i, 0)),
            pl.BlockSpec((NCORE, RBLK), lambda i: (0, i)),
            pl.BlockSpec((1, D), lambda i: (0, 0)),
        ],
        out_specs=[
            pl.BlockSpec((RBLK, D), lambda i: (i, 0)),
            pl.BlockSpec((RBLK, D), lambda i: (i, 0)),
        ],
        out_shape=[
            jax.ShapeDtypeStruct((N, D), jnp.float32),
            jax.ShapeDtypeStruct((N, D), jnp.float32),
        ],
    )(q, y2, degp, b2)


def kernel(x, edge_index, W1, b1, W2, b2):
    src = edge_index[0].astype(jnp.int32)
    dst = edge_index[1].astype(jnp.int32)
    pad_e = EP - E
    pad_src = jnp.arange(pad_e, dtype=jnp.int32) % N
    src_p = jnp.concatenate([src, pad_src])
    pad_dst = N + (jnp.arange(pad_e, dtype=jnp.int32) % (NP - N))
    dst_p = jnp.concatenate([dst, pad_dst])
    dst128 = dst_p.reshape(EP // CH, CH)
    src_p = src_p.reshape(EP // ACH, ACH)
    dst_p = dst_p.reshape(EP // ACH, ACH)
    zvec = jnp.zeros((ROWS_PT,), jnp.float32)
    zrows = jnp.zeros((ROWS_PT, D), jnp.float32)

    degp = _sc_degree(dst128, zvec)
    y1 = _tc_stage_a(x, W1, degp)
    p = _sc_aggregate(y1, src_p, dst_p, zrows)
    y2 = _tc_stage_c(p, y1, degp, W2, b1.reshape(1, D))
    q = _sc_aggregate(y2, src_p, dst_p, zrows)
    out, logp = _tc_stage_e(q, y2, degp, b2.reshape(1, D))
    return (out, logp)


# speedup vs baseline: 3.2099x; 1.0090x over previous
import functools

import jax
import jax.numpy as jnp
from jax import lax
from jax.experimental import pallas as pl
from jax.experimental.pallas import tpu as pltpu
from jax.experimental.pallas import tpu_sc as plsc

N = 10000
NP = 10240
E = 320000
EP = 327680
D = 128
NSUB = 16
NCORE = 2
EPW = EP // (NSUB * NCORE)
CH = 128
NCH = EPW // CH
ROWS_PT = NP // NSUB
RBLK = 8192
GRID = (N + RBLK - 1) // RBLK


def _sc_mesh():
    return plsc.VectorSubcoreMesh(core_axis_name="c", subcore_axis_name="s")


def _sc_degree(dst_p, zvec):
    @functools.partial(
        pl.kernel,
        out_type=jax.ShapeDtypeStruct((NCORE, NP), jnp.float32),
        mesh=_sc_mesh(),
        scratch_types=[
            pltpu.VMEM_SHARED((NP,), jnp.float32),
            pltpu.VMEM((NCH, CH), jnp.int32),
            pltpu.VMEM((CH,), jnp.float32),
        ],
    )
    def k(dst_hbm, z_hbm, out_hbm, dacc, dstv, ones_v):
        cid = lax.axis_index("c")
        sid = lax.axis_index("s")
        gw = cid * NSUB + sid
        for j in range(CH // 16):
            ones_v[pl.ds(j * 16, 16)] = jnp.full((16,), 1.0, jnp.float32)
        pltpu.sync_copy(dst_hbm.at[pl.ds(gw * NCH, NCH)], dstv)
        pltpu.sync_copy(z_hbm, dacc.at[pl.ds(sid * ROWS_PT, ROWS_PT)])
        plsc.subcore_barrier()

        def body(c, carry):
            pltpu.sync_copy(ones_v, dacc.at[dstv.at[c]], add=True)
            return carry

        lax.fori_loop(0, NCH, body, 0)
        plsc.subcore_barrier()
        pltpu.sync_copy(dacc.at[pl.ds(sid * ROWS_PT, ROWS_PT)],
                        out_hbm.at[cid, pl.ds(sid * ROWS_PT, ROWS_PT)])

    return k(dst_p, zvec)


ACH = 64
ANCH = EPW // ACH
AH = ANCH // 4
NPRE = 4


def _sc_aggregate(y, src_p, dst_p, zrows):
    @functools.partial(
        pl.kernel,
        out_type=jax.ShapeDtypeStruct((NCORE, NP, D), jnp.float32),
        mesh=_sc_mesh(),
        scratch_types=[
            pltpu.VMEM_SHARED((NP, D), jnp.float32),
            pltpu.VMEM((AH, ACH), jnp.int32),
            pltpu.VMEM((AH, ACH), jnp.int32),
            pltpu.VMEM((4, ACH, D), jnp.float32),
            pltpu.SemaphoreType.DMA((4,)),
            pltpu.SemaphoreType.DMA((4,)),
        ],
    )
    def k(y_hbm, src_hbm, dst_hbm, z_hbm, out_hbm, acc, srcv, dstv, rows,
          gs, ss):
        cid = lax.axis_index("c")
        sid = lax.axis_index("s")
        gw = cid * NSUB + sid
        pltpu.sync_copy(z_hbm, acc.at[pl.ds(sid * ROWS_PT, ROWS_PT)])
        plsc.subcore_barrier()

        def gstart(c, b):
            pltpu.async_copy(y_hbm.at[srcv.at[c]], rows.at[b], gs.at[b])

        def gwait(c, b):
            pltpu.make_async_copy(y_hbm.at[srcv.at[c]], rows.at[b],
                                  gs.at[b]).wait()

        def sstart(c, b):
            pltpu.async_copy(rows.at[b], acc.at[dstv.at[c]], ss.at[b],
                             add=True)

        def swait(c, b):
            pltpu.make_async_copy(rows.at[b], acc.at[dstv.at[c]],
                                  ss.at[b]).wait()

        def step(c, b):
            gwait(c, b)
            sstart(c, b)
            swait(c - 2, (b + 2) & 3)
            gstart(c + 2, (b + 2) & 3)

        for h in range(NPRE):
            pltpu.sync_copy(src_hbm.at[pl.ds(gw * ANCH + h * AH, AH)], srcv)
            pltpu.sync_copy(dst_hbm.at[pl.ds(gw * ANCH + h * AH, AH)], dstv)
            gstart(0, 0)
            gstart(1, 1)
            gwait(0, 0); sstart(0, 0); gstart(2, 2)
            gwait(1, 1); sstart(1, 1); gstart(3, 3)

            def body(i, carry):
                c0 = 4 * i + 2
                step(c0 + 0, 2)
                step(c0 + 1, 3)
                step(c0 + 2, 0)
                step(c0 + 3, 1)
                return carry

            lax.fori_loop(0, (AH - 4) // 4, body, 0)
            gwait(AH - 2, 2); sstart(AH - 2, 2)
            gwait(AH - 1, 3); sstart(AH - 1, 3)
            swait(AH - 4, 0)
            swait(AH - 3, 1)
            swait(AH - 2, 2)
            swait(AH - 1, 3)

        plsc.subcore_barrier()
        pltpu.sync_copy(acc.at[pl.ds(sid * ROWS_PT, ROWS_PT)],
                        out_hbm.at[cid, pl.ds(sid * ROWS_PT, ROWS_PT)])

    return k(y, src_p, dst_p, zrows)


def _dinv_from(dp_ref):
    deg = dp_ref[0, :] + dp_ref[1, :] + 1.0
    return lax.rsqrt(deg)[:, None]


def _tc_stage_a(x_p, W1, degp):
    def body(x_ref, w_ref, dp_ref, y_ref):
        xw = jnp.dot(x_ref[...], w_ref[...],
                     preferred_element_type=jnp.float32)
        y_ref[...] = xw * _dinv_from(dp_ref)

    return pl.pallas_call(
        body,
        grid=(GRID,),
        in_specs=[
            pl.BlockSpec((RBLK, D), lambda i: (i, 0)),
            pl.BlockSpec((D, D), lambda i: (0, 0)),
            pl.BlockSpec((NCORE, RBLK), lambda i: (0, i)),
        ],
        out_specs=pl.BlockSpec((RBLK, D), lambda i: (i, 0)),
        out_shape=jax.ShapeDtypeStruct((N, D), jnp.float32),
    )(x_p, W1, degp)


def _tc_stage_c(p, y1, degp, W2, b1):
    def body(p_ref, y_ref, dp_ref, w_ref, b_ref, o_ref):
        dinv = _dinv_from(dp_ref)
        acc = p_ref[0] + p_ref[1] + y_ref[...]
        h = jnp.maximum(acc * dinv + b_ref[...], 0.0)
        o_ref[...] = jnp.dot(h, w_ref[...],
                             preferred_element_type=jnp.float32) * dinv

    return pl.pallas_call(
        body,
        grid=(GRID,),
        in_specs=[
            pl.BlockSpec((NCORE, RBLK, D), lambda i: (0, i, 0)),
            pl.BlockSpec((RBLK, D), lambda i: (i, 0)),
            pl.BlockSpec((NCORE, RBLK), lambda i: (0, i)),
            pl.BlockSpec((D, D), lambda i: (0, 0)),
            pl.BlockSpec((1, D), lambda i: (0, 0)),
        ],
        out_specs=pl.BlockSpec((RBLK, D), lambda i: (i, 0)),
        out_shape=jax.ShapeDtypeStruct((N, D), jnp.float32),
    )(p, y1, degp, W2, b1)


def _tc_stage_e(q, y2, degp, b2):
    def body(q_ref, y_ref, dp_ref, b_ref, o_ref, l_ref):
        dinv = _dinv_from(dp_ref)
        out = (q_ref[0] + q_ref[1] + y_ref[...]) * dinv + b_ref[...]
        m = jnp.max(out, axis=1, keepdims=True)
        ex = jnp.exp(out - m)
        s = jnp.sum(ex, axis=1, keepdims=True)
        o_ref[...] = out
        l_ref[...] = out - m - jnp.log(s)

    return pl.pallas_call(
        body,
        grid=(GRID,),
        in_specs=[
            pl.BlockSpec((NCORE, RBLK, D), lambda i: (0, i, 0)),
            pl.BlockSpec((RBLK, D), lambda i: (i, 0)),
            pl.BlockSpec((NCORE, RBLK), lambda i: (0, i)),
            pl.BlockSpec((1, D), lambda i: (0, 0)),
        ],
        out_specs=[
            pl.BlockSpec((RBLK, D), lambda i: (i, 0)),
            pl.BlockSpec((RBLK, D), lambda i: (i, 0)),
        ],
        out_shape=[
            jax.ShapeDtypeStruct((N, D), jnp.float32),
            jax.ShapeDtypeStruct((N, D), jnp.float32),
        ],
    )(q, y2, degp, b2)


def kernel(x, edge_index, W1, b1, W2, b2):
    src = edge_index[0].astype(jnp.int32)
    dst = edge_index[1].astype(jnp.int32)
    pad_e = EP - E
    pad_src = jnp.arange(pad_e, dtype=jnp.int32) % N
    src_p = jnp.concatenate([src, pad_src])
    pad_dst = N + (jnp.arange(pad_e, dtype=jnp.int32) % (NP - N))
    dst_p = jnp.concatenate([dst, pad_dst])
    dst128 = dst_p.reshape(EP // CH, CH)
    src_p = src_p.reshape(EP // ACH, ACH)
    dst_p = dst_p.reshape(EP // ACH, ACH)
    zvec = jnp.zeros((ROWS_PT,), jnp.float32)
    zrows = jnp.zeros((ROWS_PT, D), jnp.float32)

    degp = _sc_degree(dst128, zvec)
    y1 = _tc_stage_a(x, W1, degp)
    p = _sc_aggregate(y1, src_p, dst_p, zrows)
    y2 = _tc_stage_c(p, y1, degp, W2, b1.reshape(1, D))
    q = _sc_aggregate(y2, src_p, dst_p, zrows)
    out, logp = _tc_stage_e(q, y2, degp, b2.reshape(1, D))
    return (out, logp)
